# Initial kernel scaffold; baseline (speedup 1.0000x reference)
import jax
import jax.numpy as jnp
from jax.experimental import pallas as pl


def kernel(story, C):
    raise NotImplementedError("write your pallas kernel here")


# speedup vs baseline: 8.9796x; 8.9796x over previous
import functools

import jax
import jax.numpy as jnp
from jax import lax
from jax.experimental import pallas as pl
from jax.experimental.pallas import tpu as pltpu
from jax.experimental.pallas import tpu_sc as plsc

NC, NS = 2, 16
NW = NC * NS
LANES = 16
GROWS = 128


def _sc_gather_sums(story2d, cflat, *, V, d, M, n_slots):
    slots_w = n_slots // NW
    rows_w = slots_w * M // GROWS
    spg = GROWS // M
    n_sec = 4
    gps = rows_w // n_sec
    sec_slots = slots_w // n_sec
    mesh = plsc.VectorSubcoreMesh(
        core_axis_name="c", subcore_axis_name="s",
        num_cores=NC, num_subcores=NS)

    @functools.partial(
        pl.kernel,
        out_type=jax.ShapeDtypeStruct((3, n_slots, d), jnp.float32),
        mesh=mesh,
        scratch_types=[
            pltpu.VMEM((rows_w, GROWS), jnp.int32),
            pltpu.VMEM((GROWS, d), jnp.float32),
            pltpu.VMEM((sec_slots, d), jnp.float32),
        ],
        compiler_params=pltpu.CompilerParams(use_tc_tiling_on_sc=False),
    )
    def k(story_ref, cflat_ref, m_ref, idx_v, rows_v, out_v):
        wid = lax.axis_index("s") * NC + lax.axis_index("c")
        pltpu.sync_copy(story_ref.at[pl.ds(wid * rows_w, rows_w)], idx_v)

        def table_pass(t, carry):
            def add_v(g, c):
                for i in range(GROWS // LANES):
                    sl = pl.ds(i * LANES, LANES)
                    idx_v[g, sl] = idx_v[g, sl] + V
                return c
            lax.fori_loop(0, rows_w, add_v, 0)

            def section(h, c):
                def gath(q, cc):
                    pltpu.sync_copy(cflat_ref.at[idx_v.at[h * gps + q]],
                                    rows_v)
                    for s8 in range(spg):
                        for jj in range(d // LANES):
                            sl = pl.ds(jj * LANES, LANES)
                            acc = rows_v[s8 * M, sl]
                            for mm in range(1, M):
                                acc = acc + rows_v[s8 * M + mm, sl]
                            out_v[q * spg + s8, sl] = acc
                    return cc
                lax.fori_loop(0, gps, gath, 0)
                pltpu.sync_copy(
                    out_v,
                    m_ref.at[t, pl.ds(wid * slots_w + h * sec_slots,
                                      sec_slots)])
                return c
            lax.fori_loop(0, n_sec, section, 0)
            return carry
        lax.fori_loop(0, 3, table_pass, 0)

    return k(story2d, cflat)


def _tc_attention(m, *, B, S, d, BB=128):
    inv_s = 1.0 / S

    def body(m_ref, u_ref):
        m1 = m_ref[0]
        m2 = m_ref[1]
        m3 = m_ref[2]
        u1 = jnp.sum(m1, axis=1) * inv_s
        p1 = jax.nn.softmax(jnp.sum(m1 * u1[:, None, :], axis=2), axis=1)
        u2 = u1 + jnp.sum(m2 * p1[:, :, None], axis=1)
        p2 = jax.nn.softmax(jnp.sum(m2 * u2[:, None, :], axis=2), axis=1)
        u3 = u2 + jnp.sum(m3 * p2[:, :, None], axis=1)
        u_ref[...] = u3

    return pl.pallas_call(
        body,
        grid=(B // BB,),
        in_specs=[pl.BlockSpec((3, BB, S, d), lambda i: (0, i, 0, 0))],
        out_specs=pl.BlockSpec((BB, d), lambda i: (i, 0)),
        out_shape=jax.ShapeDtypeStruct((B, d), jnp.float32),
    )(m)


def kernel(story, C):
    S, B, M = story.shape
    V, d = C.shape[1], C.shape[2]
    n_slots = B * S
    st = jnp.transpose(story.astype(jnp.int32), (1, 0, 2))
    story2d = st.reshape(n_slots * M // GROWS, GROWS)
    cflat = C.reshape(C.shape[0] * V, d)
    m = _sc_gather_sums(story2d, cflat, V=V, d=d, M=M, n_slots=n_slots)
    u = _tc_attention(m.reshape(3, B, S, d), B=B, S=S, d=d)
    return u


# speedup vs baseline: 9.2332x; 1.0282x over previous
import functools

import jax
import jax.numpy as jnp
from jax import lax
from jax.experimental import pallas as pl
from jax.experimental.pallas import tpu as pltpu
from jax.experimental.pallas import tpu_sc as plsc

NC, NS = 2, 16
NW = NC * NS
LANES = 16
GROWS = 128


def _sc_gather_sums(story2d, cflat, *, V, d, M, n_slots):
    slots_w = n_slots // NW
    rows_w = slots_w * M // GROWS
    spg = GROWS // M
    n_sec = 4
    gps = rows_w // n_sec
    sec_slots = slots_w // n_sec
    mesh = plsc.VectorSubcoreMesh(
        core_axis_name="c", subcore_axis_name="s",
        num_cores=NC, num_subcores=NS)

    @functools.partial(
        pl.kernel,
        out_type=jax.ShapeDtypeStruct((3, n_slots, d), jnp.float32),
        mesh=mesh,
        scratch_types=[
            pltpu.VMEM((rows_w, GROWS), jnp.int32),
            pltpu.VMEM((2, GROWS, d), jnp.float32),
            pltpu.VMEM((sec_slots, d), jnp.float32),
            pltpu.SemaphoreType.DMA,
            pltpu.SemaphoreType.DMA,
        ],
        compiler_params=pltpu.CompilerParams(use_tc_tiling_on_sc=False),
    )
    def k(story_ref, cflat_ref, m_ref, idx_v, rows_v, out_v, sem0, sem1):
        wid = lax.axis_index("s") * NC + lax.axis_index("c")
        pltpu.sync_copy(story_ref.at[pl.ds(wid * rows_w, rows_w)], idx_v)
        sems = (sem0, sem1)

        def fire(row, p):
            pltpu.async_copy(cflat_ref.at[idx_v.at[row]], rows_v.at[p],
                             sems[p])

        def drain(p):
            pltpu.make_async_copy(cflat_ref.at[idx_v.at[0]], rows_v.at[p],
                                  sems[p]).wait()

        def compute(p, q):
            for s8 in range(spg):
                for jj in range(d // LANES):
                    sl = pl.ds(jj * LANES, LANES)
                    acc = rows_v[p, s8 * M, sl]
                    for mm in range(1, M):
                        acc = acc + rows_v[p, s8 * M + mm, sl]
                    out_v[q * spg + s8, sl] = acc

        def table_pass(t, carry):
            def add_v(g, c):
                for i in range(GROWS // LANES):
                    sl = pl.ds(i * LANES, LANES)
                    idx_v[g, sl] = idx_v[g, sl] + V
                return c
            lax.fori_loop(0, rows_w, add_v, 0)

            def section(h, c):
                base = h * gps
                fire(base, 0)
                fire(base + 1, 1)

                def pair(j, cc):
                    q0 = 2 * j
                    drain(0)
                    compute(0, q0)

                    @pl.when(j < gps // 2 - 1)
                    def _():
                        fire(base + q0 + 2, 0)
                    drain(1)
                    compute(1, q0 + 1)

                    @pl.when(j < gps // 2 - 1)
                    def _():
                        fire(base + q0 + 3, 1)
                    return cc
                lax.fori_loop(0, gps // 2, pair, 0)
                pltpu.sync_copy(
                    out_v,
                    m_ref.at[t, pl.ds(wid * slots_w + h * sec_slots,
                                      sec_slots)])
                return c
            lax.fori_loop(0, n_sec, section, 0)
            return carry
        lax.fori_loop(0, 3, table_pass, 0)

    return k(story2d, cflat)


def _tc_attention(m, *, B, S, d, BB=128):
    inv_s = 1.0 / S

    def body(m_ref, u_ref):
        m1 = m_ref[0]
        m2 = m_ref[1]
        m3 = m_ref[2]
        u1 = jnp.sum(m1, axis=1) * inv_s
        p1 = jax.nn.softmax(jnp.sum(m1 * u1[:, None, :], axis=2), axis=1)
        u2 = u1 + jnp.sum(m2 * p1[:, :, None], axis=1)
        p2 = jax.nn.softmax(jnp.sum(m2 * u2[:, None, :], axis=2), axis=1)
        u3 = u2 + jnp.sum(m3 * p2[:, :, None], axis=1)
        u_ref[...] = u3

    return pl.pallas_call(
        body,
        grid=(B // BB,),
        in_specs=[pl.BlockSpec((3, BB, S, d), lambda i: (0, i, 0, 0))],
        out_specs=pl.BlockSpec((BB, d), lambda i: (i, 0)),
        out_shape=jax.ShapeDtypeStruct((B, d), jnp.float32),
    )(m)


def kernel(story, C):
    S, B, M = story.shape
    V, d = C.shape[1], C.shape[2]
    n_slots = B * S
    st = jnp.transpose(story.astype(jnp.int32), (1, 0, 2))
    story2d = st.reshape(n_slots * M // GROWS, GROWS)
    cflat = C.reshape(C.shape[0] * V, d)
    m = _sc_gather_sums(story2d, cflat, V=V, d=d, M=M, n_slots=n_slots)
    u = _tc_attention(m.reshape(3, B, S, d), B=B, S=S, d=d)
    return u
